# baseline (speedup 1.0000x reference)
import functools
import math

import jax
import jax.numpy as jnp
from jax import lax
from jax.experimental import pallas as pl
from jax.experimental.pallas import tpu as pltpu
from jax.experimental.pallas import tpu_sc as plsc

NC = 2
NS = 16
NW = NC * NS
K = 128


def _ceil_to(a, m):
  return (a + m - 1) // m * m


def _make_deg_kernel(NPad, EPW):
  CH = EPW // K
  W = min(8, CH)
  RPS = NPad // NS
  mesh = plsc.VectorSubcoreMesh(core_axis_name="c", subcore_axis_name="s",
                                num_cores=NC)

  @functools.partial(
      pl.kernel, mesh=mesh,
      out_type=jax.ShapeDtypeStruct((NC * NPad,), jnp.float32),
      scratch_types=[
          pltpu.VMEM((CH, K), jnp.int32),
          pltpu.VMEM((CH, K), jnp.float32),
          pltpu.VMEM((K,), jnp.float32),
          pltpu.VMEM_SHARED((NPad,), jnp.float32),
          pltpu.SemaphoreType.DMA,
          pltpu.SemaphoreType.DMA,
      ],
  )
  def deg_kernel(col_hbm, ew_hbm, out_hbm, colb, ewb, zv, acc_sh, bsem, ssem):
    cid = lax.axis_index("c")
    sid = lax.axis_index("s")
    wid = sid * NC + cid
    cb = wid * CH

    pltpu.async_copy(col_hbm.at[pl.ds(cb, CH)], colb, bsem)
    pltpu.async_copy(ew_hbm.at[pl.ds(cb, CH)], ewb, bsem)
    for j in range(K // 16):
      zv[pl.ds(j * 16, 16)] = jnp.zeros((16,), jnp.float32)
    for u in range(RPS // K):
      pltpu.sync_copy(zv, acc_sh.at[pl.ds(sid * RPS + u * K, K)])
    plsc.subcore_barrier()
    pltpu.make_async_copy(col_hbm.at[pl.ds(cb, CH)], colb, bsem).wait()
    pltpu.make_async_copy(ew_hbm.at[pl.ds(cb, CH)], ewb, bsem).wait()

    def chunk(i, carry):
      pltpu.async_copy(ewb.at[i], acc_sh.at[colb.at[i]], ssem, add=True)

      @pl.when(i >= W)
      def _():
        pltpu.make_async_copy(ewb.at[i - W], acc_sh.at[colb.at[i - W]],
                              ssem).wait()

      return carry

    lax.fori_loop(0, CH, chunk, 0)
    for u in range(W):
      pltpu.make_async_copy(ewb.at[CH - W + u], acc_sh.at[colb.at[CH - W + u]],
                            ssem).wait()
    plsc.subcore_barrier()
    pltpu.sync_copy(acc_sh.at[pl.ds(sid * RPS, RPS)],
                    out_hbm.at[pl.ds(cid * NPad + sid * RPS, RPS)])

  return deg_kernel


NSET = 4
KM = 80
BG = 8
MSG_CORES = 1


def _make_msg_kernel(NPad, D, EPW):
  CH = EPW // KM
  NGRP = CH // BG
  DV = D // 16
  RPS = NPad // NS
  mesh = plsc.VectorSubcoreMesh(core_axis_name="c", subcore_axis_name="s",
                                num_cores=MSG_CORES)

  scratch = ([pltpu.VMEM((2 * BG, 2, KM), jnp.int32),
              pltpu.VMEM((2 * BG + 1, KM), jnp.float32)] +
             [pltpu.VMEM((KM, D), jnp.float32)] * NSET +
             [pltpu.VMEM_SHARED((NPad, D), jnp.float32)] +
             [pltpu.SemaphoreType.DMA] * (2 * NSET + 2))

  @functools.partial(
      pl.kernel, mesh=mesh,
      out_type=jax.ShapeDtypeStruct((MSG_CORES * NPad, D), jnp.float32),
      scratch_types=scratch,
  )
  def msg_kernel(h_hbm, ed_hbm, ewm_hbm, out_hbm, edb, ewb, *bufs):
    rows = bufs[0:NSET]
    acc_sh = bufs[NSET]
    gs = bufs[NSET + 1:2 * NSET + 1]
    ss = bufs[2 * NSET + 1:3 * NSET + 1]
    bs_ed, bs_ew = bufs[3 * NSET + 1:3 * NSET + 3]
    cid = lax.axis_index("c")
    sid = lax.axis_index("s")
    ngrp = NGRP
    cbase = sid * CH

    def scale_rows(brow, rows_v):
      def scale(e, c2):
        s = ewb[brow, pl.ds(e, 16)][0]
        for j in range(DV):
          rows_v[e, pl.ds(j * 16, 16)] = rows_v[e, pl.ds(j * 16, 16)] * s
        return c2

      lax.fori_loop(0, KM, scale, 0)

    def gather(brow, s):
      pltpu.async_copy(h_hbm.at[edb.at[brow, 0]], rows[s], gs[s])

    def wait_gather(brow, s):
      pltpu.make_async_copy(h_hbm.at[edb.at[brow, 0]], rows[s], gs[s]).wait()

    def scatter(brow, s):
      pltpu.async_copy(rows[s], acc_sh.at[edb.at[brow, 1]], ss[s], add=True)

    def wait_scatter(brow, s):
      pltpu.make_async_copy(rows[s], acc_sh.at[edb.at[brow, 1]], ss[s]).wait()

    def zero_row(e, carry):
      for j in range(DV):
        rows[0][e, pl.ds(j * 16, 16)] = jnp.zeros((16,), jnp.float32)
      return carry

    lax.fori_loop(0, KM, zero_row, 0)
    for u in range(RPS // KM):
      pltpu.sync_copy(rows[0], acc_sh.at[pl.ds(sid * RPS + u * KM, KM)])
    plsc.subcore_barrier()

    pltpu.sync_copy(ed_hbm.at[pl.ds(cbase, BG)], edb.at[pl.ds(0, BG)])
    pltpu.sync_copy(ewm_hbm.at[pl.ds(cbase, BG)], ewb.at[pl.ds(0, BG)])
    gather(0, 0)
    gather(1, 1)

    def group(g, carry):
      par = g % 2
      bi = par * BG
      obi = (1 - par) * BG
      c0 = g * BG
      for lc in range(BG):
        s = lc % NSET
        c = c0 + lc
        brow = bi + lc
        if lc == 2:
          @pl.when(g + 1 < ngrp)
          def _():
            nb = cbase + c0 + BG
            pltpu.async_copy(ed_hbm.at[pl.ds(nb, BG)],
                             edb.at[pl.ds(obi, BG)], bs_ed)
            pltpu.async_copy(ewm_hbm.at[pl.ds(nb, BG)],
                             ewb.at[pl.ds(obi, BG)], bs_ew)

        if lc == 6:
          @pl.when(g + 1 < ngrp)
          def _():
            nb = cbase + c0 + BG
            pltpu.make_async_copy(ed_hbm.at[pl.ds(nb, BG)],
                                  edb.at[pl.ds(obi, BG)], bs_ed).wait()
            pltpu.make_async_copy(ewm_hbm.at[pl.ds(nb, BG)],
                                  ewb.at[pl.ds(obi, BG)], bs_ew).wait()

        wait_gather(brow, s)
        scale_rows(brow, rows[s])
        scatter(brow, s)

        pf = (s + 2) % NSET
        if lc < 6:
          @pl.when(c >= 2)
          def _():
            drow = bi + lc - 2 if lc >= 2 else obi + BG + lc - 2
            wait_scatter(drow, pf)

          gather(bi + lc + 2, pf)
        else:
          @pl.when(g + 1 < ngrp)
          def _():
            wait_scatter(bi + lc - 2, pf)
            gather(obi + lc - 6, pf)

      return carry

    lax.fori_loop(0, ngrp, group, 0)
    biL = ((ngrp - 1) % 2) * BG
    for u in range(4):
      wait_scatter(biL + BG - 4 + u, u)
    plsc.subcore_barrier()
    pltpu.sync_copy(acc_sh.at[pl.ds(sid * RPS, RPS)],
                    out_hbm.at[pl.ds(cid * NPad + sid * RPS, RPS)])

  return msg_kernel


def _dinv_from(d_ref):
  deg = d_ref[:, 0:1] + d_ref[:, 1:2] + 1.0
  return jnp.where(deg > 0, lax.rsqrt(deg), 0.0)


def _dot(a, b):
  return lax.dot_general(a, b, (((1,), (0,)), ((), ())),
                         precision=lax.Precision.HIGHEST,
                         preferred_element_type=jnp.float32)


def _tc_stage1(xp, W1, RB):
  NPad, D = xp.shape
  G = NPad // RB

  def body(x_ref, w_ref, o_ref):
    o_ref[...] = _dot(x_ref[...], w_ref[...])

  return pl.pallas_call(
      body, grid=(G,),
      in_specs=[
          pl.BlockSpec((RB, D), lambda i: (i, 0)),
          pl.BlockSpec((D, W1.shape[1]), lambda i: (0, 0)),
      ],
      out_specs=pl.BlockSpec((RB, W1.shape[1]), lambda i: (i, 0)),
      out_shape=jax.ShapeDtypeStruct((NPad, W1.shape[1]), jnp.float32),
  )(xp, W1)


def _tc_stage1b(H1, degT, RB):
  NPad, D = H1.shape
  G = NPad // RB

  def body(h_ref, d_ref, o_ref):
    o_ref[...] = h_ref[...] * _dinv_from(d_ref)

  return pl.pallas_call(
      body, grid=(G,),
      in_specs=[
          pl.BlockSpec((RB, D), lambda i: (i, 0)),
          pl.BlockSpec((RB, 2), lambda i: (i, 0)),
      ],
      out_specs=pl.BlockSpec((RB, D), lambda i: (i, 0)),
      out_shape=jax.ShapeDtypeStruct((NPad, D), jnp.float32),
  )(H1, degT)


def _tc_stage2(P1, H1p, degT, b1, W2, RB):
  NPad, D = H1p.shape
  G = NPad // RB

  NCP = P1.shape[0]

  def body(p_ref, h_ref, d_ref, b_ref, w_ref, o_ref):
    dinv = _dinv_from(d_ref)
    psum = p_ref[0] if NCP == 1 else p_ref[0] + p_ref[1]
    z = jnp.maximum(dinv * (psum + h_ref[...]) + b_ref[...], 0.0)
    o_ref[...] = _dot(z, w_ref[...]) * dinv

  return pl.pallas_call(
      body, grid=(G,),
      in_specs=[
          pl.BlockSpec((NCP, RB, D), lambda i: (0, i, 0)),
          pl.BlockSpec((RB, D), lambda i: (i, 0)),
          pl.BlockSpec((RB, 2), lambda i: (i, 0)),
          pl.BlockSpec((1, D), lambda i: (0, 0)),
          pl.BlockSpec((D, W2.shape[1]), lambda i: (0, 0)),
      ],
      out_specs=pl.BlockSpec((RB, W2.shape[1]), lambda i: (i, 0)),
      out_shape=jax.ShapeDtypeStruct((NPad, W2.shape[1]), jnp.float32),
  )(P1, H1p, degT, b1, W2)


def _tc_stage3(P2, H2p, degT, b2, Wc, bc, RB):
  NPad, D = H2p.shape
  C = Wc.shape[1]
  G = NPad // RB

  NCP = P2.shape[0]

  def body(p_ref, h_ref, d_ref, b_ref, w_ref, bc_ref, o_ref):
    dinv = _dinv_from(d_ref)
    psum = p_ref[0] if NCP == 1 else p_ref[0] + p_ref[1]
    z = jnp.maximum(dinv * (psum + h_ref[...]) + b_ref[...], 0.0)
    o_ref[...] = _dot(z, w_ref[...]) + bc_ref[...]

  return pl.pallas_call(
      body, grid=(G,),
      in_specs=[
          pl.BlockSpec((NCP, RB, D), lambda i: (0, i, 0)),
          pl.BlockSpec((RB, D), lambda i: (i, 0)),
          pl.BlockSpec((RB, 2), lambda i: (i, 0)),
          pl.BlockSpec((1, D), lambda i: (0, 0)),
          pl.BlockSpec((D, C), lambda i: (0, 0)),
          pl.BlockSpec((1, C), lambda i: (0, 0)),
      ],
      out_specs=pl.BlockSpec((RB, C), lambda i: (i, 0)),
      out_shape=jax.ShapeDtypeStruct((NPad, C), jnp.float32),
  )(P2, H2p, degT, b2, Wc, bc)


def kernel(x, edge_index, edge_weight, W1, b1, W2, b2, Wc, bc):
  N, D = x.shape
  E = edge_index.shape[1]

  EPW = _ceil_to(max(E // NW, 1), math.lcm(K, NSET * KM))
  Epad = EPW * NW
  pe = Epad - E
  row = jnp.concatenate([edge_index[0], jnp.zeros((pe,), jnp.int32)])
  col = jnp.concatenate([edge_index[1], jnp.zeros((pe,), jnp.int32)])
  ew = jnp.concatenate([edge_weight, jnp.zeros((pe,), jnp.float32)])
  CHT = Epad // KM
  ed = jnp.stack([row.reshape(CHT, KM), col.reshape(CHT, KM)], axis=1)
  ewm = ew.reshape(CHT, KM)

  NPad = _ceil_to(N, NS * math.lcm(K, KM))
  xp = jnp.concatenate([x, jnp.zeros((NPad - N, D), x.dtype)], axis=0)

  RB = 1024 if NPad % 1024 == 0 else K

  CHD = Epad // K
  deg_parts = _make_deg_kernel(NPad, EPW)(col.reshape(CHD, K),
                                          ew.reshape(CHD, K))
  degT = jnp.transpose(deg_parts.reshape(NC, NPad))

  H1 = _tc_stage1(xp, W1, RB)
  H1p = _tc_stage1b(H1, degT, RB)

  EPW_MSG = Epad // (MSG_CORES * NS)

  P1 = _make_msg_kernel(NPad, D, EPW_MSG)(H1p, ed, ewm)
  P1 = P1.reshape(MSG_CORES, NPad, D)

  H2p = _tc_stage2(P1, H1p, degT, b1.reshape(1, -1), W2, RB)
  P2 = _make_msg_kernel(NPad, D, EPW_MSG)(H2p, ed, ewm)
  P2 = P2.reshape(MSG_CORES, NPad, D)

  out = _tc_stage3(P2, H2p, degT, b2.reshape(1, -1), Wc,
                   bc.reshape(1, -1), RB)
  return out[:N]


# speedup vs baseline: 1.4097x; 1.4097x over previous
import functools
import math

import jax
import jax.numpy as jnp
from jax import lax
from jax.experimental import pallas as pl
from jax.experimental.pallas import tpu as pltpu
from jax.experimental.pallas import tpu_sc as plsc

NC = 2
NS = 16
NW = NC * NS
K = 128


def _ceil_to(a, m):
  return (a + m - 1) // m * m


def _make_deg_kernel(NPad, EPW):
  CH = EPW // K
  W = min(8, CH)
  RPS = NPad // NS
  mesh = plsc.VectorSubcoreMesh(core_axis_name="c", subcore_axis_name="s",
                                num_cores=NC)

  @functools.partial(
      pl.kernel, mesh=mesh,
      out_type=jax.ShapeDtypeStruct((NC * NPad,), jnp.float32),
      scratch_types=[
          pltpu.VMEM((CH, K), jnp.int32),
          pltpu.VMEM((CH, K), jnp.float32),
          pltpu.VMEM((K,), jnp.float32),
          pltpu.VMEM_SHARED((NPad,), jnp.float32),
          pltpu.SemaphoreType.DMA,
          pltpu.SemaphoreType.DMA,
      ],
  )
  def deg_kernel(col_hbm, ew_hbm, out_hbm, colb, ewb, zv, acc_sh, bsem, ssem):
    cid = lax.axis_index("c")
    sid = lax.axis_index("s")
    wid = sid * NC + cid
    cb = wid * CH

    pltpu.async_copy(col_hbm.at[pl.ds(cb, CH)], colb, bsem)
    pltpu.async_copy(ew_hbm.at[pl.ds(cb, CH)], ewb, bsem)
    for j in range(K // 16):
      zv[pl.ds(j * 16, 16)] = jnp.zeros((16,), jnp.float32)
    for u in range(RPS // K):
      pltpu.sync_copy(zv, acc_sh.at[pl.ds(sid * RPS + u * K, K)])
    plsc.subcore_barrier()
    pltpu.make_async_copy(col_hbm.at[pl.ds(cb, CH)], colb, bsem).wait()
    pltpu.make_async_copy(ew_hbm.at[pl.ds(cb, CH)], ewb, bsem).wait()

    def chunk(i, carry):
      pltpu.async_copy(ewb.at[i], acc_sh.at[colb.at[i]], ssem, add=True)

      @pl.when(i >= W)
      def _():
        pltpu.make_async_copy(ewb.at[i - W], acc_sh.at[colb.at[i - W]],
                              ssem).wait()

      return carry

    lax.fori_loop(0, CH, chunk, 0)
    for u in range(W):
      pltpu.make_async_copy(ewb.at[CH - W + u], acc_sh.at[colb.at[CH - W + u]],
                            ssem).wait()
    plsc.subcore_barrier()
    pltpu.sync_copy(acc_sh.at[pl.ds(sid * RPS, RPS)],
                    out_hbm.at[pl.ds(cid * NPad + sid * RPS, RPS)])

  return deg_kernel


NSET = 4
KM = 80
BG = 8
SPLIT_A = 0.97


def _make_msg_kernel(NPad, D, EPW_A, EPW_B):
  CH_A = EPW_A // KM
  CH_B = EPW_B // KM
  NG_A = CH_A // BG
  NG_B = CH_B // BG
  DV = D // 16
  RPS = NPad // NS
  mesh = plsc.VectorSubcoreMesh(core_axis_name="c", subcore_axis_name="s",
                                num_cores=NC)

  scratch = ([pltpu.VMEM((2 * BG, 2, KM), jnp.int32),
              pltpu.VMEM((2 * BG + 1, KM), jnp.float32)] +
             [pltpu.VMEM((KM, D), jnp.float32)] * NSET +
             [pltpu.VMEM_SHARED((NPad, D), jnp.float32)] +
             [pltpu.SemaphoreType.DMA] * (2 * NSET + 2))

  @functools.partial(
      pl.kernel, mesh=mesh,
      out_type=jax.ShapeDtypeStruct((NC * NPad, D), jnp.float32),
      scratch_types=scratch,
  )
  def msg_kernel(h_hbm, ed_hbm, ewm_hbm, out_hbm, edb, ewb, *bufs):
    rows = bufs[0:NSET]
    acc_sh = bufs[NSET]
    gs = bufs[NSET + 1:2 * NSET + 1]
    ss = bufs[2 * NSET + 1:3 * NSET + 1]
    bs_ed, bs_ew = bufs[3 * NSET + 1:3 * NSET + 3]
    cid = lax.axis_index("c")
    sid = lax.axis_index("s")
    is_a = cid == 0
    ngrp = jnp.where(is_a, NG_A, NG_B)
    ch = jnp.where(is_a, CH_A, CH_B)
    cbase = jnp.where(is_a, sid * CH_A, NS * CH_A + sid * CH_B)

    def scale_rows(brow, rows_v):
      def scale(e, c2):
        s = ewb[brow, pl.ds(e, 16)][0]
        for j in range(DV):
          rows_v[e, pl.ds(j * 16, 16)] = rows_v[e, pl.ds(j * 16, 16)] * s
        return c2

      lax.fori_loop(0, KM, scale, 0)

    def gather(brow, s):
      pltpu.async_copy(h_hbm.at[edb.at[brow, 0]], rows[s], gs[s])

    def wait_gather(brow, s):
      pltpu.make_async_copy(h_hbm.at[edb.at[brow, 0]], rows[s], gs[s]).wait()

    def scatter(brow, s):
      pltpu.async_copy(rows[s], acc_sh.at[edb.at[brow, 1]], ss[s], add=True)

    def wait_scatter(brow, s):
      pltpu.make_async_copy(rows[s], acc_sh.at[edb.at[brow, 1]], ss[s]).wait()

    def zero_row(e, carry):
      for j in range(DV):
        rows[0][e, pl.ds(j * 16, 16)] = jnp.zeros((16,), jnp.float32)
      return carry

    lax.fori_loop(0, KM, zero_row, 0)
    for u in range(RPS // KM):
      pltpu.sync_copy(rows[0], acc_sh.at[pl.ds(sid * RPS + u * KM, KM)])
    plsc.subcore_barrier()

    pltpu.sync_copy(ed_hbm.at[pl.ds(cbase, BG)], edb.at[pl.ds(0, BG)])
    pltpu.sync_copy(ewm_hbm.at[pl.ds(cbase, BG)], ewb.at[pl.ds(0, BG)])
    gather(0, 0)
    gather(1, 1)

    def group(g, carry):
      par = g % 2
      bi = par * BG
      obi = (1 - par) * BG
      c0 = g * BG
      for lc in range(BG):
        s = lc % NSET
        c = c0 + lc
        brow = bi + lc
        if lc == 2:
          @pl.when(g + 1 < ngrp)
          def _():
            nb = cbase + c0 + BG
            pltpu.async_copy(ed_hbm.at[pl.ds(nb, BG)],
                             edb.at[pl.ds(obi, BG)], bs_ed)
            pltpu.async_copy(ewm_hbm.at[pl.ds(nb, BG)],
                             ewb.at[pl.ds(obi, BG)], bs_ew)

        if lc == 6:
          @pl.when(g + 1 < ngrp)
          def _():
            nb = cbase + c0 + BG
            pltpu.make_async_copy(ed_hbm.at[pl.ds(nb, BG)],
                                  edb.at[pl.ds(obi, BG)], bs_ed).wait()
            pltpu.make_async_copy(ewm_hbm.at[pl.ds(nb, BG)],
                                  ewb.at[pl.ds(obi, BG)], bs_ew).wait()

        wait_gather(brow, s)
        scale_rows(brow, rows[s])
        scatter(brow, s)

        pf = (s + 2) % NSET
        if lc < 6:
          @pl.when(c >= 2)
          def _():
            drow = bi + lc - 2 if lc >= 2 else obi + BG + lc - 2
            wait_scatter(drow, pf)

          gather(bi + lc + 2, pf)
        else:
          @pl.when(g + 1 < ngrp)
          def _():
            wait_scatter(bi + lc - 2, pf)
            gather(obi + lc - 6, pf)

      return carry

    lax.fori_loop(0, ngrp, group, 0)
    biL = ((ngrp - 1) % 2) * BG
    for u in range(4):
      wait_scatter(biL + BG - 4 + u, u)
    plsc.subcore_barrier()
    pltpu.sync_copy(acc_sh.at[pl.ds(sid * RPS, RPS)],
                    out_hbm.at[pl.ds(cid * NPad + sid * RPS, RPS)])

  return msg_kernel


def _dinv_from(d_ref):
  deg = d_ref[:, 0:1] + d_ref[:, 1:2] + 1.0
  return jnp.where(deg > 0, lax.rsqrt(deg), 0.0)


def _dot(a, b):
  return lax.dot_general(a, b, (((1,), (0,)), ((), ())),
                         precision=lax.Precision.HIGHEST,
                         preferred_element_type=jnp.float32)


def _tc_stage1(xp, W1, RB):
  NPad, D = xp.shape
  G = NPad // RB

  def body(x_ref, w_ref, o_ref):
    o_ref[...] = _dot(x_ref[...], w_ref[...])

  return pl.pallas_call(
      body, grid=(G,),
      in_specs=[
          pl.BlockSpec((RB, D), lambda i: (i, 0)),
          pl.BlockSpec((D, W1.shape[1]), lambda i: (0, 0)),
      ],
      out_specs=pl.BlockSpec((RB, W1.shape[1]), lambda i: (i, 0)),
      out_shape=jax.ShapeDtypeStruct((NPad, W1.shape[1]), jnp.float32),
  )(xp, W1)


def _tc_stage1b(H1, degT, RB):
  NPad, D = H1.shape
  G = NPad // RB

  def body(h_ref, d_ref, o_ref):
    o_ref[...] = h_ref[...] * _dinv_from(d_ref)

  return pl.pallas_call(
      body, grid=(G,),
      in_specs=[
          pl.BlockSpec((RB, D), lambda i: (i, 0)),
          pl.BlockSpec((RB, 2), lambda i: (i, 0)),
      ],
      out_specs=pl.BlockSpec((RB, D), lambda i: (i, 0)),
      out_shape=jax.ShapeDtypeStruct((NPad, D), jnp.float32),
  )(H1, degT)


def _tc_stage2(P1, H1p, degT, b1, W2, RB):
  NPad, D = H1p.shape
  G = NPad // RB

  def body(p_ref, h_ref, d_ref, b_ref, w_ref, o_ref):
    dinv = _dinv_from(d_ref)
    z = jnp.maximum(dinv * (p_ref[0] + p_ref[1] + h_ref[...]) + b_ref[...],
                    0.0)
    o_ref[...] = _dot(z, w_ref[...]) * dinv

  return pl.pallas_call(
      body, grid=(G,),
      in_specs=[
          pl.BlockSpec((2, RB, D), lambda i: (0, i, 0)),
          pl.BlockSpec((RB, D), lambda i: (i, 0)),
          pl.BlockSpec((RB, 2), lambda i: (i, 0)),
          pl.BlockSpec((1, D), lambda i: (0, 0)),
          pl.BlockSpec((D, W2.shape[1]), lambda i: (0, 0)),
      ],
      out_specs=pl.BlockSpec((RB, W2.shape[1]), lambda i: (i, 0)),
      out_shape=jax.ShapeDtypeStruct((NPad, W2.shape[1]), jnp.float32),
  )(P1, H1p, degT, b1, W2)


def _tc_stage3(P2, H2p, degT, b2, Wc, bc, RB):
  NPad, D = H2p.shape
  C = Wc.shape[1]
  G = NPad // RB

  def body(p_ref, h_ref, d_ref, b_ref, w_ref, bc_ref, o_ref):
    dinv = _dinv_from(d_ref)
    z = jnp.maximum(dinv * (p_ref[0] + p_ref[1] + h_ref[...]) + b_ref[...],
                    0.0)
    o_ref[...] = _dot(z, w_ref[...]) + bc_ref[...]

  return pl.pallas_call(
      body, grid=(G,),
      in_specs=[
          pl.BlockSpec((2, RB, D), lambda i: (0, i, 0)),
          pl.BlockSpec((RB, D), lambda i: (i, 0)),
          pl.BlockSpec((RB, 2), lambda i: (i, 0)),
          pl.BlockSpec((1, D), lambda i: (0, 0)),
          pl.BlockSpec((D, C), lambda i: (0, 0)),
          pl.BlockSpec((1, C), lambda i: (0, 0)),
      ],
      out_specs=pl.BlockSpec((RB, C), lambda i: (i, 0)),
      out_shape=jax.ShapeDtypeStruct((NPad, C), jnp.float32),
  )(P2, H2p, degT, b2, Wc, bc)


def kernel(x, edge_index, edge_weight, W1, b1, W2, b2, Wc, bc):
  N, D = x.shape
  E = edge_index.shape[1]

  EPW = _ceil_to(max(E // NW, 1), math.lcm(K, NSET * KM))
  Epad = EPW * NW
  pe = Epad - E
  row = jnp.concatenate([edge_index[0], jnp.zeros((pe,), jnp.int32)])
  col = jnp.concatenate([edge_index[1], jnp.zeros((pe,), jnp.int32)])
  ew = jnp.concatenate([edge_weight, jnp.zeros((pe,), jnp.float32)])
  CHT = Epad // KM
  ed = jnp.stack([row.reshape(CHT, KM), col.reshape(CHT, KM)], axis=1)
  ewm = ew.reshape(CHT, KM)

  NPad = _ceil_to(N, NS * math.lcm(K, KM))
  xp = jnp.concatenate([x, jnp.zeros((NPad - N, D), x.dtype)], axis=0)

  RB = 1024 if NPad % 1024 == 0 else K

  CHD = Epad // K
  deg_parts = _make_deg_kernel(NPad, EPW)(col.reshape(CHD, K),
                                          ew.reshape(CHD, K))
  degT = jnp.transpose(deg_parts.reshape(NC, NPad))

  H1 = _tc_stage1(xp, W1, RB)
  H1p = _tc_stage1b(H1, degT, RB)

  unit = BG * KM
  U = (Epad // NS) // unit
  UA = max(1, min(U - 1, int(round(U * SPLIT_A))))
  EPW_A = UA * unit
  EPW_B = (U - UA) * unit

  P1 = _make_msg_kernel(NPad, D, EPW_A, EPW_B)(H1p, ed, ewm)
  P1 = P1.reshape(NC, NPad, D)

  H2p = _tc_stage2(P1, H1p, degT, b1.reshape(1, -1), W2, RB)
  P2 = _make_msg_kernel(NPad, D, EPW_A, EPW_B)(H2p, ed, ewm)
  P2 = P2.reshape(NC, NPad, D)

  out = _tc_stage3(P2, H2p, degT, b2.reshape(1, -1), Wc,
                   bc.reshape(1, -1), RB)
  return out[:N]


# speedup vs baseline: 1.5478x; 1.0980x over previous
import functools
import math

import jax
import jax.numpy as jnp
from jax import lax
from jax.experimental import pallas as pl
from jax.experimental.pallas import tpu as pltpu
from jax.experimental.pallas import tpu_sc as plsc

NC = 2
NS = 16
NW = NC * NS
K = 128


def _ceil_to(a, m):
  return (a + m - 1) // m * m


def _make_deg_kernel(NPad, EPW):
  CH = EPW // K
  W = min(8, CH)
  RPS = NPad // NS
  mesh = plsc.VectorSubcoreMesh(core_axis_name="c", subcore_axis_name="s",
                                num_cores=NC)

  @functools.partial(
      pl.kernel, mesh=mesh,
      out_type=jax.ShapeDtypeStruct((NC * NPad,), jnp.float32),
      scratch_types=[
          pltpu.VMEM((CH, K), jnp.int32),
          pltpu.VMEM((CH, K), jnp.float32),
          pltpu.VMEM((K,), jnp.float32),
          pltpu.VMEM_SHARED((NPad,), jnp.float32),
          pltpu.SemaphoreType.DMA,
          pltpu.SemaphoreType.DMA,
      ],
  )
  def deg_kernel(col_hbm, ew_hbm, out_hbm, colb, ewb, zv, acc_sh, bsem, ssem):
    cid = lax.axis_index("c")
    sid = lax.axis_index("s")
    wid = sid * NC + cid
    cb = wid * CH

    pltpu.async_copy(col_hbm.at[pl.ds(cb, CH)], colb, bsem)
    pltpu.async_copy(ew_hbm.at[pl.ds(cb, CH)], ewb, bsem)
    for j in range(K // 16):
      zv[pl.ds(j * 16, 16)] = jnp.zeros((16,), jnp.float32)
    for u in range(RPS // K):
      pltpu.sync_copy(zv, acc_sh.at[pl.ds(sid * RPS + u * K, K)])
    plsc.subcore_barrier()
    pltpu.make_async_copy(col_hbm.at[pl.ds(cb, CH)], colb, bsem).wait()
    pltpu.make_async_copy(ew_hbm.at[pl.ds(cb, CH)], ewb, bsem).wait()

    def chunk(i, carry):
      pltpu.async_copy(ewb.at[i], acc_sh.at[colb.at[i]], ssem, add=True)

      @pl.when(i >= W)
      def _():
        pltpu.make_async_copy(ewb.at[i - W], acc_sh.at[colb.at[i - W]],
                              ssem).wait()

      return carry

    lax.fori_loop(0, CH, chunk, 0)
    for u in range(W):
      pltpu.make_async_copy(ewb.at[CH - W + u], acc_sh.at[colb.at[CH - W + u]],
                            ssem).wait()
    plsc.subcore_barrier()
    pltpu.sync_copy(acc_sh.at[pl.ds(sid * RPS, RPS)],
                    out_hbm.at[pl.ds(cid * NPad + sid * RPS, RPS)])

  return deg_kernel


NSET = 4
KM = 80
BG = 8
SPLIT_A = 0.91


def _make_msg_kernel(NPad, D, EPW_A, EPW_B):
  CH_A = EPW_A // KM
  CH_B = EPW_B // KM
  NG_A = CH_A // BG
  NG_B = CH_B // BG
  DV = D // 16
  RPS = NPad // NS
  mesh = plsc.VectorSubcoreMesh(core_axis_name="c", subcore_axis_name="s",
                                num_cores=NC)

  scratch = ([pltpu.VMEM((2 * BG, 2, KM), jnp.int32),
              pltpu.VMEM((2 * BG + 1, KM), jnp.float32)] +
             [pltpu.VMEM((KM, D), jnp.float32)] * NSET +
             [pltpu.VMEM_SHARED((NPad, D), jnp.float32)] +
             [pltpu.SemaphoreType.DMA] * (2 * NSET + 2))

  @functools.partial(
      pl.kernel, mesh=mesh,
      out_type=jax.ShapeDtypeStruct((NC * NPad, D), jnp.float32),
      scratch_types=scratch,
  )
  def msg_kernel(h_hbm, ed_hbm, ewm_hbm, out_hbm, edb, ewb, *bufs):
    rows = bufs[0:NSET]
    acc_sh = bufs[NSET]
    gs = bufs[NSET + 1:2 * NSET + 1]
    ss = bufs[2 * NSET + 1:3 * NSET + 1]
    bs_ed, bs_ew = bufs[3 * NSET + 1:3 * NSET + 3]
    cid = lax.axis_index("c")
    sid = lax.axis_index("s")
    is_a = cid == 0
    ngrp = jnp.where(is_a, NG_A, NG_B)
    ch = jnp.where(is_a, CH_A, CH_B)
    cbase = jnp.where(is_a, sid * CH_A, NS * CH_A + sid * CH_B)

    def scale_rows(brow, rows_v):
      def scale(e, c2):
        s = ewb[brow, pl.ds(e, 16)][0]
        for j in range(DV):
          rows_v[e, pl.ds(j * 16, 16)] = rows_v[e, pl.ds(j * 16, 16)] * s
        return c2

      lax.fori_loop(0, KM, scale, 0)

    def gather(brow, s):
      pltpu.async_copy(h_hbm.at[edb.at[brow, 0]], rows[s], gs[s])

    def wait_gather(brow, s):
      pltpu.make_async_copy(h_hbm.at[edb.at[brow, 0]], rows[s], gs[s]).wait()

    def scatter(brow, s):
      pltpu.async_copy(rows[s], acc_sh.at[edb.at[brow, 1]], ss[s], add=True)

    def wait_scatter(brow, s):
      pltpu.make_async_copy(rows[s], acc_sh.at[edb.at[brow, 1]], ss[s]).wait()

    def zero_row(e, carry):
      for j in range(DV):
        rows[0][e, pl.ds(j * 16, 16)] = jnp.zeros((16,), jnp.float32)
      return carry

    lax.fori_loop(0, KM, zero_row, 0)
    for u in range(RPS // KM):
      pltpu.sync_copy(rows[0], acc_sh.at[pl.ds(sid * RPS + u * KM, KM)])
    plsc.subcore_barrier()

    pltpu.sync_copy(ed_hbm.at[pl.ds(cbase, BG)], edb.at[pl.ds(0, BG)])
    pltpu.sync_copy(ewm_hbm.at[pl.ds(cbase, BG)], ewb.at[pl.ds(0, BG)])
    gather(0, 0)
    gather(1, 1)

    def group(g, carry):
      par = g % 2
      bi = par * BG
      obi = (1 - par) * BG
      c0 = g * BG
      for lc in range(BG):
        s = lc % NSET
        c = c0 + lc
        brow = bi + lc
        if lc == 2:
          @pl.when(g + 1 < ngrp)
          def _():
            nb = cbase + c0 + BG
            pltpu.async_copy(ed_hbm.at[pl.ds(nb, BG)],
                             edb.at[pl.ds(obi, BG)], bs_ed)
            pltpu.async_copy(ewm_hbm.at[pl.ds(nb, BG)],
                             ewb.at[pl.ds(obi, BG)], bs_ew)

        if lc == 6:
          @pl.when(g + 1 < ngrp)
          def _():
            nb = cbase + c0 + BG
            pltpu.make_async_copy(ed_hbm.at[pl.ds(nb, BG)],
                                  edb.at[pl.ds(obi, BG)], bs_ed).wait()
            pltpu.make_async_copy(ewm_hbm.at[pl.ds(nb, BG)],
                                  ewb.at[pl.ds(obi, BG)], bs_ew).wait()

        wait_gather(brow, s)
        scale_rows(brow, rows[s])
        scatter(brow, s)

        pf = (s + 2) % NSET
        if lc < 6:
          @pl.when(c >= 2)
          def _():
            drow = bi + lc - 2 if lc >= 2 else obi + BG + lc - 2
            wait_scatter(drow, pf)

          gather(bi + lc + 2, pf)
        else:
          @pl.when(g + 1 < ngrp)
          def _():
            wait_scatter(bi + lc - 2, pf)
            gather(obi + lc - 6, pf)

      return carry

    lax.fori_loop(0, ngrp, group, 0)
    biL = ((ngrp - 1) % 2) * BG
    for u in range(4):
      wait_scatter(biL + BG - 4 + u, u)
    plsc.subcore_barrier()
    pltpu.sync_copy(acc_sh.at[pl.ds(sid * RPS, RPS)],
                    out_hbm.at[pl.ds(cid * NPad + sid * RPS, RPS)])

  return msg_kernel


def _dinv_from(d_ref):
  deg = d_ref[:, 0:1] + d_ref[:, 1:2] + 1.0
  return jnp.where(deg > 0, lax.rsqrt(deg), 0.0)


def _dot(a, b):
  return lax.dot_general(a, b, (((1,), (0,)), ((), ())),
                         precision=lax.Precision.HIGHEST,
                         preferred_element_type=jnp.float32)


def _tc_stage1(xp, W1, RB):
  NPad, D = xp.shape
  G = NPad // RB

  def body(x_ref, w_ref, o_ref):
    o_ref[...] = _dot(x_ref[...], w_ref[...])

  return pl.pallas_call(
      body, grid=(G,),
      in_specs=[
          pl.BlockSpec((RB, D), lambda i: (i, 0)),
          pl.BlockSpec((D, W1.shape[1]), lambda i: (0, 0)),
      ],
      out_specs=pl.BlockSpec((RB, W1.shape[1]), lambda i: (i, 0)),
      out_shape=jax.ShapeDtypeStruct((NPad, W1.shape[1]), jnp.float32),
  )(xp, W1)


def _tc_stage1b(H1, degT, RB):
  NPad, D = H1.shape
  G = NPad // RB

  def body(h_ref, d_ref, o_ref):
    o_ref[...] = h_ref[...] * _dinv_from(d_ref)

  return pl.pallas_call(
      body, grid=(G,),
      in_specs=[
          pl.BlockSpec((RB, D), lambda i: (i, 0)),
          pl.BlockSpec((RB, 2), lambda i: (i, 0)),
      ],
      out_specs=pl.BlockSpec((RB, D), lambda i: (i, 0)),
      out_shape=jax.ShapeDtypeStruct((NPad, D), jnp.float32),
  )(H1, degT)


def _tc_stage2(P1, H1p, degT, b1, W2, RB):
  NPad, D = H1p.shape
  G = NPad // RB

  def body(p_ref, h_ref, d_ref, b_ref, w_ref, o_ref):
    dinv = _dinv_from(d_ref)
    z = jnp.maximum(dinv * (p_ref[0] + p_ref[1] + h_ref[...]) + b_ref[...],
                    0.0)
    o_ref[...] = _dot(z, w_ref[...]) * dinv

  return pl.pallas_call(
      body, grid=(G,),
      in_specs=[
          pl.BlockSpec((2, RB, D), lambda i: (0, i, 0)),
          pl.BlockSpec((RB, D), lambda i: (i, 0)),
          pl.BlockSpec((RB, 2), lambda i: (i, 0)),
          pl.BlockSpec((1, D), lambda i: (0, 0)),
          pl.BlockSpec((D, W2.shape[1]), lambda i: (0, 0)),
      ],
      out_specs=pl.BlockSpec((RB, W2.shape[1]), lambda i: (i, 0)),
      out_shape=jax.ShapeDtypeStruct((NPad, W2.shape[1]), jnp.float32),
  )(P1, H1p, degT, b1, W2)


def _tc_stage3(P2, H2p, degT, b2, Wc, bc, RB):
  NPad, D = H2p.shape
  C = Wc.shape[1]
  G = NPad // RB

  def body(p_ref, h_ref, d_ref, b_ref, w_ref, bc_ref, o_ref):
    dinv = _dinv_from(d_ref)
    z = jnp.maximum(dinv * (p_ref[0] + p_ref[1] + h_ref[...]) + b_ref[...],
                    0.0)
    o_ref[...] = _dot(z, w_ref[...]) + bc_ref[...]

  return pl.pallas_call(
      body, grid=(G,),
      in_specs=[
          pl.BlockSpec((2, RB, D), lambda i: (0, i, 0)),
          pl.BlockSpec((RB, D), lambda i: (i, 0)),
          pl.BlockSpec((RB, 2), lambda i: (i, 0)),
          pl.BlockSpec((1, D), lambda i: (0, 0)),
          pl.BlockSpec((D, C), lambda i: (0, 0)),
          pl.BlockSpec((1, C), lambda i: (0, 0)),
      ],
      out_specs=pl.BlockSpec((RB, C), lambda i: (i, 0)),
      out_shape=jax.ShapeDtypeStruct((NPad, C), jnp.float32),
  )(P2, H2p, degT, b2, Wc, bc)


def kernel(x, edge_index, edge_weight, W1, b1, W2, b2, Wc, bc):
  N, D = x.shape
  E = edge_index.shape[1]

  EPW = _ceil_to(max(E // NW, 1), math.lcm(K, NSET * KM))
  Epad = EPW * NW
  pe = Epad - E
  row = jnp.concatenate([edge_index[0], jnp.zeros((pe,), jnp.int32)])
  col = jnp.concatenate([edge_index[1], jnp.zeros((pe,), jnp.int32)])
  ew = jnp.concatenate([edge_weight, jnp.zeros((pe,), jnp.float32)])
  CHT = Epad // KM
  ed = jnp.stack([row.reshape(CHT, KM), col.reshape(CHT, KM)], axis=1)
  ewm = ew.reshape(CHT, KM)

  NPad = _ceil_to(N, NS * math.lcm(K, KM))
  xp = jnp.concatenate([x, jnp.zeros((NPad - N, D), x.dtype)], axis=0)

  RB = 1024 if NPad % 1024 == 0 else K

  CHD = Epad // K
  deg_parts = _make_deg_kernel(NPad, EPW)(col.reshape(CHD, K),
                                          ew.reshape(CHD, K))
  degT = jnp.transpose(deg_parts.reshape(NC, NPad))

  H1 = _tc_stage1(xp, W1, RB)
  H1p = _tc_stage1b(H1, degT, RB)

  unit = BG * KM
  U = (Epad // NS) // unit
  UA = max(1, min(U - 1, int(round(U * SPLIT_A))))
  EPW_A = UA * unit
  EPW_B = (U - UA) * unit

  P1 = _make_msg_kernel(NPad, D, EPW_A, EPW_B)(H1p, ed, ewm)
  P1 = P1.reshape(NC, NPad, D)

  H2p = _tc_stage2(P1, H1p, degT, b1.reshape(1, -1), W2, RB)
  P2 = _make_msg_kernel(NPad, D, EPW_A, EPW_B)(H2p, ed, ewm)
  P2 = P2.reshape(NC, NPad, D)

  out = _tc_stage3(P2, H2p, degT, b2.reshape(1, -1), Wc,
                   bc.reshape(1, -1), RB)
  return out[:N]
